# Initial kernel scaffold; baseline (speedup 1.0000x reference)
import jax
import jax.numpy as jnp
from jax.experimental import pallas as pl


def kernel(x, M, V, ln_g, ln_b, mW1, mb1, mW2, mb2, nW1, nb1, nW2, nb2, iW1, ib1, iW2, ib2, gW, gb, edge_index):
    raise NotImplementedError("write your pallas kernel here")


# speedup vs baseline: 4.7681x; 4.7681x over previous
import functools

import jax
import jax.numpy as jnp
from jax import lax
from jax.experimental import pallas as pl
from jax.experimental.pallas import tpu as pltpu
from jax.experimental.pallas import tpu_sc as plsc

NC = 2
NS = 16
LANES = 16


def _table_body(v_ref, t_ref):
    v = v_ref[...]
    n = jnp.sqrt(v[:, 0:16] ** 2 + v[:, 16:32] ** 2 + v[:, 32:48] ** 2)
    t_ref[...] = jnp.concatenate([v, n], axis=1)


def _node_table(v2):
    n = v2.shape[0]
    return pl.pallas_call(
        _table_body,
        out_shape=jax.ShapeDtypeStruct((n, 64), jnp.float32),
    )(v2)


def _make_sc_gather(n_nodes, n_edges):
    nwk = NC * NS
    ew = n_edges // nwk
    chunk = 80
    n_chunks = ew // chunk
    mesh = plsc.VectorSubcoreMesh(core_axis_name="c", subcore_axis_name="s", num_cores=NC, num_subcores=NS)

    @functools.partial(
        pl.kernel,
        out_type=[
            jax.ShapeDtypeStruct((n_edges, 64), jnp.float32),
            jax.ShapeDtypeStruct((n_edges, 64), jnp.float32),
        ],
        mesh=mesh,
        scratch_types=[
            pltpu.VMEM((chunk,), jnp.int32),
            pltpu.VMEM((chunk,), jnp.int32),
            pltpu.VMEM((chunk, 64), jnp.float32),
            pltpu.VMEM((chunk, 64), jnp.float32),
            pltpu.SemaphoreType.DMA,
            pltpu.SemaphoreType.DMA,
        ],
        compiler_params=pltpu.CompilerParams(use_tc_tiling_on_sc=False),
    )
    def k(t_hbm, row_hbm, col_hbm, gr_hbm, gc_hbm, idxr, idxc, gr_v, gc_v,
          sem1, sem2):
        wid = lax.axis_index("s") * NC + lax.axis_index("c")
        base = wid * ew

        def body(i, carry):
            off = base + i * chunk
            pltpu.sync_copy(row_hbm.at[pl.ds(off, chunk)], idxr)
            pltpu.sync_copy(col_hbm.at[pl.ds(off, chunk)], idxc)
            cp1 = pltpu.async_copy(t_hbm.at[idxr], gr_v, sem1)
            cp2 = pltpu.async_copy(t_hbm.at[idxc], gc_v, sem2)
            cp1.wait()
            cp2.wait()
            pltpu.sync_copy(gr_v, gr_hbm.at[pl.ds(off, chunk)])
            pltpu.sync_copy(gc_v, gc_hbm.at[pl.ds(off, chunk)])
            return carry

        lax.fori_loop(0, n_chunks, body, 0)

    return k


def _edge_body(m_ref, gr_ref, gc_ref, ln_g_ref, ln_b_ref, w1_ref, b1_ref,
               w2_ref, b2_ref, iw1_ref, ib1_ref, iw2_ref, ib2_ref,
               mo_ref, wm_ref):
    m = m_ref[...]
    mu = jnp.mean(m, axis=-1, keepdims=True)
    var = jnp.mean((m - mu) ** 2, axis=-1, keepdims=True)
    mn = (m - mu) * lax.rsqrt(var + 1e-5) * ln_g_ref[...] + ln_b_ref[...]
    h = jax.nn.relu(
        jnp.dot(mn, w1_ref[...], preferred_element_type=jnp.float32)
        + b1_ref[...])
    mo = mn + jnp.dot(h, w2_ref[...], preferred_element_type=jnp.float32) \
        + b2_ref[...]

    gr = gr_ref[...]
    gc = gc_ref[...]
    ns = gr[:, 48:64]
    nd = gc[:, 48:64]
    dot = (gr[:, 0:16] * gc[:, 0:16] + gr[:, 16:32] * gc[:, 16:32]
           + gr[:, 32:48] * gc[:, 32:48])
    cos = dot / (ns * nd + 1e-8)
    inv = jnp.concatenate([ns, nd, cos], axis=1)
    ih = jax.nn.relu(
        jnp.dot(inv, iw1_ref[...], preferred_element_type=jnp.float32)
        + ib1_ref[...])
    alpha = jax.nn.sigmoid(
        jnp.sum(ih * iw2_ref[...], axis=1, keepdims=True) + ib2_ref[...])
    mo_ref[...] = mo
    wm_ref[...] = mo * alpha


def _tc_edge(m, gr, gc, ln_g, ln_b, w1, b1, w2, b2, iw1, ib1, iw2t, ib2):
    e, d = m.shape
    be = 2000
    grid = (e // be,)
    full = lambda shp: pl.BlockSpec(shp, lambda i: (0, 0))
    return pl.pallas_call(
        _edge_body,
        grid=grid,
        in_specs=[
            pl.BlockSpec((be, d), lambda i: (i, 0)),
            pl.BlockSpec((be, 64), lambda i: (i, 0)),
            pl.BlockSpec((be, 64), lambda i: (i, 0)),
            full((1, d)), full((1, d)),
            full((d, d)), full((1, d)),
            full((d, d)), full((1, d)),
            full((48, d)), full((1, d)),
            full((1, d)), full((1, 1)),
        ],
        out_specs=[
            pl.BlockSpec((be, d), lambda i: (i, 0)),
            pl.BlockSpec((be, d), lambda i: (i, 0)),
        ],
        out_shape=[
            jax.ShapeDtypeStruct((e, d), jnp.float32),
            jax.ShapeDtypeStruct((e, d), jnp.float32),
        ],
    )(m, gr, gc, ln_g, ln_b, w1, b1, w2, b2, iw1, ib1, iw2t, ib2)


def _make_sc_scatter(n_nodes, n_edges, d):
    nwk = NC * NS
    ew = n_edges // nwk
    chunk = 80
    n_chunks = ew // chunk
    rows_per_tile = n_nodes // NS
    zrows = 125
    nz = rows_per_tile // zrows
    mesh = plsc.VectorSubcoreMesh(core_axis_name="c", subcore_axis_name="s", num_cores=NC, num_subcores=NS)

    @functools.partial(
        pl.kernel,
        out_type=jax.ShapeDtypeStruct((NC * n_nodes, d), jnp.float32),
        mesh=mesh,
        scratch_types=[
            pltpu.VMEM((chunk,), jnp.int32),
            pltpu.VMEM((chunk, d), jnp.float32),
            pltpu.VMEM((zrows, d), jnp.float32),
            pltpu.VMEM_SHARED((n_nodes, d), jnp.float32),
        ],
        compiler_params=pltpu.CompilerParams(use_tc_tiling_on_sc=False),
    )
    def k(w_hbm, row_hbm, out_hbm, idx, wv, zbuf, accum):
        c = lax.axis_index("c")
        s = lax.axis_index("s")
        wid = s * NC + c
        base = wid * ew

        def zb(i, carry):
            for j in range(d // LANES):
                zbuf[i, pl.ds(j * LANES, LANES)] = jnp.zeros(
                    (LANES,), jnp.float32)
            return carry

        lax.fori_loop(0, zrows, zb, 0)
        for p in range(nz):
            pltpu.sync_copy(
                zbuf, accum.at[pl.ds(s * rows_per_tile + p * zrows, zrows)])
        plsc.subcore_barrier()

        def body(i, carry):
            off = base + i * chunk
            pltpu.sync_copy(row_hbm.at[pl.ds(off, chunk)], idx)
            pltpu.sync_copy(w_hbm.at[pl.ds(off, chunk)], wv)
            pltpu.sync_copy(wv, accum.at[idx], add=True)
            return carry

        lax.fori_loop(0, n_chunks, body, 0)
        plsc.subcore_barrier()

        pltpu.sync_copy(
            accum.at[pl.ds(s * rows_per_tile, rows_per_tile)],
            out_hbm.at[pl.ds(c * n_nodes + s * rows_per_tile,
                             rows_per_tile)])

    return k


def _node_body(p_ref, x_ref, v_ref, nw1_ref, nb1_ref, nw2_ref, nb2_ref,
               gw_ref, gb_ref, xo_ref, vo_ref):
    m = p_ref[0] + p_ref[1]
    nh = jax.nn.relu(
        jnp.dot(m, nw1_ref[...], preferred_element_type=jnp.float32)
        + nb1_ref[...])
    xo = x_ref[...] + jnp.dot(nh, nw2_ref[...],
                              preferred_element_type=jnp.float32) \
        + nb2_ref[...]
    g = jax.nn.sigmoid(
        jnp.dot(xo, gw_ref[...], preferred_element_type=jnp.float32)
        + gb_ref[...])
    v = v_ref[...]
    xo_ref[...] = xo
    vo_ref[...] = jnp.concatenate(
        [v[:, 0:16] * g, v[:, 16:32] * g, v[:, 32:48] * g], axis=1)


def _tc_node(p, x, v2, nw1, nb1, nw2, nb2, gw, gb):
    n, d = x.shape
    bn = 2000
    grid = (n // bn,)
    full2 = lambda shp: pl.BlockSpec(shp, lambda i: (0, 0))
    return pl.pallas_call(
        _node_body,
        grid=grid,
        in_specs=[
            pl.BlockSpec((2, bn, d), lambda i: (0, i, 0)),
            pl.BlockSpec((bn, d), lambda i: (i, 0)),
            pl.BlockSpec((bn, 48), lambda i: (i, 0)),
            full2((d, d)), full2((1, d)),
            full2((d, d)), full2((1, d)),
            full2((d, 16)), full2((1, 16)),
        ],
        out_specs=[
            pl.BlockSpec((bn, d), lambda i: (i, 0)),
            pl.BlockSpec((bn, 48), lambda i: (i, 0)),
        ],
        out_shape=[
            jax.ShapeDtypeStruct((n, d), jnp.float32),
            jax.ShapeDtypeStruct((n, 48), jnp.float32),
        ],
    )(p, x, v2, nw1, nb1, nw2, nb2, gw, gb)


def kernel(x, M, V, ln_g, ln_b, mW1, mb1, mW2, mb2, nW1, nb1, nW2, nb2,
           iW1, ib1, iW2, ib2, gW, gb, edge_index):
    n, d = x.shape
    e = M.shape[0]
    nw = V.shape[2]
    v2 = V.reshape(n, 3 * nw)
    row = edge_index[0]
    col = edge_index[1]

    t = _node_table(v2)
    gr, gc = _make_sc_gather(n, e)(t, row, col)
    m_out, wm = _tc_edge(
        M, gr, gc,
        ln_g.reshape(1, d), ln_b.reshape(1, d),
        mW1, mb1.reshape(1, d), mW2, mb2.reshape(1, d),
        iW1, ib1.reshape(1, d), iW2.reshape(1, d), ib2.reshape(1, 1))
    partials = _make_sc_scatter(n, e, d)(wm, row)
    x_out, v2_out = _tc_node(
        partials.reshape(2, n, d), x, v2,
        nW1, nb1.reshape(1, d), nW2, nb2.reshape(1, d),
        gW, gb.reshape(1, 16))
    return (x_out, m_out, v2_out.reshape(n, 3, nw))


# speedup vs baseline: 5.6340x; 1.1816x over previous
import functools

import jax
import jax.numpy as jnp
from jax import lax
from jax.experimental import pallas as pl
from jax.experimental.pallas import tpu as pltpu
from jax.experimental.pallas import tpu_sc as plsc

NC = 2
NS = 16
LANES = 16


def _table_body(v_ref, t_ref):
    v = v_ref[...]
    n = jnp.sqrt(v[:, 0:16] ** 2 + v[:, 16:32] ** 2 + v[:, 32:48] ** 2)
    t_ref[...] = jnp.concatenate([v, n], axis=1)


def _node_table(v2):
    n = v2.shape[0]
    return pl.pallas_call(
        _table_body,
        out_shape=jax.ShapeDtypeStruct((n, 64), jnp.float32),
    )(v2)


def _make_sc_gather(n_nodes, n_edges):
    nwk = NC * NS
    ew = n_edges // nwk
    chunk = 80
    n_chunks = ew // chunk
    mesh = plsc.VectorSubcoreMesh(core_axis_name="c", subcore_axis_name="s", num_cores=NC, num_subcores=NS)

    @functools.partial(
        pl.kernel,
        out_type=jax.ShapeDtypeStruct((n_edges, 128), jnp.float32),
        mesh=mesh,
        scratch_types=[
            pltpu.VMEM((chunk,), jnp.int32),
            pltpu.VMEM((chunk,), jnp.int32),
            pltpu.VMEM((chunk, 64), jnp.float32),
            pltpu.VMEM((chunk, 64), jnp.float32),
            pltpu.SemaphoreType.DMA,
            pltpu.SemaphoreType.DMA,
        ],
        compiler_params=pltpu.CompilerParams(use_tc_tiling_on_sc=False),
    )
    def k(t_hbm, row_hbm, col_hbm, g_hbm, idxr, idxc, gr_v, gc_v,
          sem1, sem2):
        wid = lax.axis_index("s") * NC + lax.axis_index("c")
        base = wid * ew

        def body(i, carry):
            off = base + i * chunk
            pltpu.sync_copy(row_hbm.at[pl.ds(off, chunk)], idxr)
            pltpu.sync_copy(col_hbm.at[pl.ds(off, chunk)], idxc)
            cp1 = pltpu.async_copy(t_hbm.at[idxr], gr_v, sem1)
            cp2 = pltpu.async_copy(t_hbm.at[idxc], gc_v, sem2)
            cp1.wait()
            cp2.wait()
            pltpu.sync_copy(gr_v, g_hbm.at[pl.ds(off, chunk), pl.ds(0, 64)])
            pltpu.sync_copy(gc_v, g_hbm.at[pl.ds(off, chunk), pl.ds(64, 64)])
            return carry

        lax.fori_loop(0, n_chunks, body, 0)

    return k


def _edge_body(m_ref, g_ref, ln_g_ref, ln_b_ref, w1_ref, b1_ref,
               w2_ref, b2_ref, iw1_ref, ib1_ref, iw2_ref, ib2_ref,
               mo_ref, wm_ref):
    m = m_ref[...]
    mu = jnp.mean(m, axis=-1, keepdims=True)
    var = jnp.mean((m - mu) ** 2, axis=-1, keepdims=True)
    mn = (m - mu) * lax.rsqrt(var + 1e-5) * ln_g_ref[...] + ln_b_ref[...]
    h = jax.nn.relu(
        jnp.dot(mn, w1_ref[...], preferred_element_type=jnp.float32)
        + b1_ref[...])
    mo = mn + jnp.dot(h, w2_ref[...], preferred_element_type=jnp.float32) \
        + b2_ref[...]

    g = g_ref[...]
    gr = g[:, 0:64]
    gc = g[:, 64:128]
    ns = gr[:, 48:64]
    nd = gc[:, 48:64]
    dot = (gr[:, 0:16] * gc[:, 0:16] + gr[:, 16:32] * gc[:, 16:32]
           + gr[:, 32:48] * gc[:, 32:48])
    cos = dot / (ns * nd + 1e-8)
    inv = jnp.concatenate([ns, nd, cos], axis=1)
    ih = jax.nn.relu(
        jnp.dot(inv, iw1_ref[...], preferred_element_type=jnp.float32)
        + ib1_ref[...])
    alpha = jax.nn.sigmoid(
        jnp.sum(ih * iw2_ref[...], axis=1, keepdims=True) + ib2_ref[...])
    mo_ref[...] = mo
    wm_ref[...] = mo * alpha


def _tc_edge(m, g, ln_g, ln_b, w1, b1, w2, b2, iw1, ib1, iw2t, ib2):
    e, d = m.shape
    be = 2000
    grid = (e // be,)
    full = lambda shp: pl.BlockSpec(shp, lambda i: (0, 0))
    return pl.pallas_call(
        _edge_body,
        grid=grid,
        in_specs=[
            pl.BlockSpec((be, d), lambda i: (i, 0)),
            pl.BlockSpec((be, 128), lambda i: (i, 0)),
            full((1, d)), full((1, d)),
            full((d, d)), full((1, d)),
            full((d, d)), full((1, d)),
            full((48, d)), full((1, d)),
            full((1, d)), full((1, 1)),
        ],
        out_specs=[
            pl.BlockSpec((be, d), lambda i: (i, 0)),
            pl.BlockSpec((be, d), lambda i: (i, 0)),
        ],
        out_shape=[
            jax.ShapeDtypeStruct((e, d), jnp.float32),
            jax.ShapeDtypeStruct((e, d), jnp.float32),
        ],
    )(m, g, ln_g, ln_b, w1, b1, w2, b2, iw1, ib1, iw2t, ib2)


def _make_sc_scatter(n_nodes, n_edges, d):
    nwk = NC * NS
    ew = n_edges // nwk
    chunk = 80
    n_chunks = ew // chunk
    rows_per_tile = n_nodes // NS
    zrows = 125
    nz = rows_per_tile // zrows
    mesh = plsc.VectorSubcoreMesh(core_axis_name="c", subcore_axis_name="s", num_cores=NC, num_subcores=NS)

    @functools.partial(
        pl.kernel,
        out_type=jax.ShapeDtypeStruct((NC * n_nodes, d), jnp.float32),
        mesh=mesh,
        scratch_types=[
            pltpu.VMEM((chunk,), jnp.int32),
            pltpu.VMEM((chunk, d), jnp.float32),
            pltpu.VMEM((zrows, d), jnp.float32),
            pltpu.VMEM_SHARED((n_nodes, d), jnp.float32),
        ],
        compiler_params=pltpu.CompilerParams(use_tc_tiling_on_sc=False),
    )
    def k(w_hbm, row_hbm, out_hbm, idx, wv, zbuf, accum):
        c = lax.axis_index("c")
        s = lax.axis_index("s")
        wid = s * NC + c
        base = wid * ew

        def zb(i, carry):
            for j in range(d // LANES):
                zbuf[i, pl.ds(j * LANES, LANES)] = jnp.zeros(
                    (LANES,), jnp.float32)
            return carry

        lax.fori_loop(0, zrows, zb, 0)
        for p in range(nz):
            pltpu.sync_copy(
                zbuf, accum.at[pl.ds(s * rows_per_tile + p * zrows, zrows)])
        plsc.subcore_barrier()

        def body(i, carry):
            off = base + i * chunk
            pltpu.sync_copy(row_hbm.at[pl.ds(off, chunk)], idx)
            pltpu.sync_copy(w_hbm.at[pl.ds(off, chunk)], wv)
            pltpu.sync_copy(wv, accum.at[idx], add=True)
            return carry

        lax.fori_loop(0, n_chunks, body, 0)
        plsc.subcore_barrier()

        pltpu.sync_copy(
            accum.at[pl.ds(s * rows_per_tile, rows_per_tile)],
            out_hbm.at[pl.ds(c * n_nodes + s * rows_per_tile,
                             rows_per_tile)])

    return k


def _node_body(p_ref, x_ref, v_ref, nw1_ref, nb1_ref, nw2_ref, nb2_ref,
               gw_ref, gb_ref, xo_ref, vo_ref):
    m = p_ref[0] + p_ref[1]
    nh = jax.nn.relu(
        jnp.dot(m, nw1_ref[...], preferred_element_type=jnp.float32)
        + nb1_ref[...])
    xo = x_ref[...] + jnp.dot(nh, nw2_ref[...],
                              preferred_element_type=jnp.float32) \
        + nb2_ref[...]
    g = jax.nn.sigmoid(
        jnp.dot(xo, gw_ref[...], preferred_element_type=jnp.float32)
        + gb_ref[...])
    v = v_ref[...]
    xo_ref[...] = xo
    vo_ref[...] = jnp.concatenate(
        [v[:, 0:16] * g, v[:, 16:32] * g, v[:, 32:48] * g], axis=1)


def _tc_node(p, x, v2, nw1, nb1, nw2, nb2, gw, gb):
    n, d = x.shape
    bn = 2000
    grid = (n // bn,)
    full2 = lambda shp: pl.BlockSpec(shp, lambda i: (0, 0))
    return pl.pallas_call(
        _node_body,
        grid=grid,
        in_specs=[
            pl.BlockSpec((2, bn, d), lambda i: (0, i, 0)),
            pl.BlockSpec((bn, d), lambda i: (i, 0)),
            pl.BlockSpec((bn, 48), lambda i: (i, 0)),
            full2((d, d)), full2((1, d)),
            full2((d, d)), full2((1, d)),
            full2((d, 16)), full2((1, 16)),
        ],
        out_specs=[
            pl.BlockSpec((bn, d), lambda i: (i, 0)),
            pl.BlockSpec((bn, 48), lambda i: (i, 0)),
        ],
        out_shape=[
            jax.ShapeDtypeStruct((n, d), jnp.float32),
            jax.ShapeDtypeStruct((n, 48), jnp.float32),
        ],
    )(p, x, v2, nw1, nb1, nw2, nb2, gw, gb)


def kernel(x, M, V, ln_g, ln_b, mW1, mb1, mW2, mb2, nW1, nb1, nW2, nb2,
           iW1, ib1, iW2, ib2, gW, gb, edge_index):
    n, d = x.shape
    e = M.shape[0]
    nw = V.shape[2]
    v2 = V.reshape(n, 3 * nw)
    row = edge_index[0]
    col = edge_index[1]

    t = _node_table(v2)
    g = _make_sc_gather(n, e)(t, row, col)
    m_out, wm = _tc_edge(
        M, g,
        ln_g.reshape(1, d), ln_b.reshape(1, d),
        mW1, mb1.reshape(1, d), mW2, mb2.reshape(1, d),
        iW1, ib1.reshape(1, d), iW2.reshape(1, d), ib2.reshape(1, 1))
    partials = _make_sc_scatter(n, e, d)(wm, row)
    x_out, v2_out = _tc_node(
        partials.reshape(2, n, d), x, v2,
        nW1, nb1.reshape(1, d), nW2, nb2.reshape(1, d),
        gW, gb.reshape(1, 16))
    return (x_out, m_out, v2_out.reshape(n, 3, nw))


# speedup vs baseline: 6.5056x; 1.1547x over previous
import functools

import jax
import jax.numpy as jnp
from jax import lax
from jax.experimental import pallas as pl
from jax.experimental.pallas import tpu as pltpu
from jax.experimental.pallas import tpu_sc as plsc

NC = 2
NS = 16
LANES = 16


def _table_body(v_ref, t_ref):
    v = v_ref[...]
    n = jnp.sqrt(v[:, 0:16] ** 2 + v[:, 16:32] ** 2 + v[:, 32:48] ** 2)
    t_ref[...] = jnp.concatenate([v, n], axis=1)


def _node_table(v2):
    n = v2.shape[0]
    return pl.pallas_call(
        _table_body,
        out_shape=jax.ShapeDtypeStruct((n, 64), jnp.float32),
    )(v2)


def _make_sc_gather(n_nodes, n_edges):
    nwk = NC * NS
    ew = n_edges // nwk
    chunk = 80
    n_chunks = ew // chunk
    mesh = plsc.VectorSubcoreMesh(core_axis_name="c", subcore_axis_name="s", num_cores=NC, num_subcores=NS)

    @functools.partial(
        pl.kernel,
        out_type=jax.ShapeDtypeStruct((n_edges, 128), jnp.float32),
        mesh=mesh,
        scratch_types=[
            pltpu.VMEM((chunk,), jnp.int32),
            pltpu.VMEM((chunk,), jnp.int32),
            pltpu.VMEM((chunk, 64), jnp.float32),
            pltpu.VMEM((chunk, 64), jnp.float32),
            pltpu.SemaphoreType.DMA,
            pltpu.SemaphoreType.DMA,
        ],
        compiler_params=pltpu.CompilerParams(use_tc_tiling_on_sc=False),
    )
    def k(t_hbm, row_hbm, col_hbm, g_hbm, idxr, idxc, gr_v, gc_v,
          sem1, sem2):
        wid = lax.axis_index("s") * NC + lax.axis_index("c")
        base = wid * ew

        def body(i, carry):
            off = base + i * chunk
            pltpu.sync_copy(row_hbm.at[pl.ds(off, chunk)], idxr)
            pltpu.sync_copy(col_hbm.at[pl.ds(off, chunk)], idxc)
            cp1 = pltpu.async_copy(t_hbm.at[idxr], gr_v, sem1)
            cp2 = pltpu.async_copy(t_hbm.at[idxc], gc_v, sem2)
            cp1.wait()
            cp2.wait()
            pltpu.sync_copy(gr_v, g_hbm.at[pl.ds(off, chunk), pl.ds(0, 64)])
            pltpu.sync_copy(gc_v, g_hbm.at[pl.ds(off, chunk), pl.ds(64, 64)])
            return carry

        lax.fori_loop(0, n_chunks, body, 0)

    return k


def _edge_body(m_ref, g_ref, ln_g_ref, ln_b_ref, w1_ref, b1_ref,
               w2_ref, b2_ref, wab_ref, crep_ref,
               ib1_ref, iw2_ref, ib2_ref, mo_ref, wm_ref):
    m = m_ref[...]
    d = m.shape[1]
    jm = jnp.full((d, d), 1.0 / d, dtype=jnp.float32)
    mu = jnp.dot(m, jm, preferred_element_type=jnp.float32)
    ms = jnp.dot(m * m, jm, preferred_element_type=jnp.float32)
    var = ms - mu * mu
    mn = (m - mu) * lax.rsqrt(var + 1e-5) * ln_g_ref[...] + ln_b_ref[...]
    h = jax.nn.relu(
        jnp.dot(mn, w1_ref[...], preferred_element_type=jnp.float32)
        + b1_ref[...])
    mo = mn + jnp.dot(h, w2_ref[...], preferred_element_type=jnp.float32) \
        + b2_ref[...]

    g = g_ref[...]
    u = g[:, 0:64] * g[:, 64:128]
    dot = u[:, 0:16] + u[:, 16:32] + u[:, 32:48]
    cos = dot / (u[:, 48:64] + 1e-8)
    ih = jax.nn.relu(
        jnp.dot(g, wab_ref[...], preferred_element_type=jnp.float32)
        + jnp.dot(cos, crep_ref[...], preferred_element_type=jnp.float32)
        + ib1_ref[...])
    alpha = jax.nn.sigmoid(
        jnp.sum(ih * iw2_ref[...], axis=1, keepdims=True) + ib2_ref[...])
    mo_ref[...] = mo
    wm_ref[...] = mo * alpha


def _tc_edge(m, g, ln_g, ln_b, w1, b1, w2, b2, iw1, ib1, iw2t, ib2):
    e, d = m.shape
    be = 2000
    grid = (e // be,)
    full = lambda shp: pl.BlockSpec(shp, lambda i: (0, 0))

    wab = jnp.zeros((d, d), jnp.float32)
    wab = wab.at[48:64].set(iw1[0:16]).at[112:128].set(iw1[16:32])
    crep = iw1[32:48]

    return pl.pallas_call(
        _edge_body,
        grid=grid,
        in_specs=[
            pl.BlockSpec((be, d), lambda i: (i, 0)),
            pl.BlockSpec((be, 128), lambda i: (i, 0)),
            full((1, d)), full((1, d)),
            full((d, d)), full((1, d)),
            full((d, d)), full((1, d)),
            full((d, d)), full((16, d)),
            full((1, d)),
            full((1, d)), full((1, 1)),
        ],
        out_specs=[
            pl.BlockSpec((be, d), lambda i: (i, 0)),
            pl.BlockSpec((be, d), lambda i: (i, 0)),
        ],
        out_shape=[
            jax.ShapeDtypeStruct((e, d), jnp.float32),
            jax.ShapeDtypeStruct((e, d), jnp.float32),
        ],
    )(m, g, ln_g, ln_b, w1, b1, w2, b2, wab, crep, ib1, iw2t, ib2)


def _make_sc_scatter(n_nodes, n_edges, d):
    nwk = NC * NS
    ew = n_edges // nwk
    chunk = 80
    n_chunks = ew // chunk
    rows_per_tile = n_nodes // NS
    zrows = 125
    nz = rows_per_tile // zrows
    mesh = plsc.VectorSubcoreMesh(core_axis_name="c", subcore_axis_name="s", num_cores=NC, num_subcores=NS)

    @functools.partial(
        pl.kernel,
        out_type=jax.ShapeDtypeStruct((NC * n_nodes, d), jnp.float32),
        mesh=mesh,
        scratch_types=[
            pltpu.VMEM((chunk,), jnp.int32),
            pltpu.VMEM((chunk, d), jnp.float32),
            pltpu.VMEM((zrows, d), jnp.float32),
            pltpu.VMEM_SHARED((n_nodes, d), jnp.float32),
        ],
        compiler_params=pltpu.CompilerParams(use_tc_tiling_on_sc=False),
    )
    def k(w_hbm, row_hbm, out_hbm, idx, wv, zbuf, accum):
        c = lax.axis_index("c")
        s = lax.axis_index("s")
        wid = s * NC + c
        base = wid * ew

        def zb(i, carry):
            for j in range(d // LANES):
                zbuf[i, pl.ds(j * LANES, LANES)] = jnp.zeros(
                    (LANES,), jnp.float32)
            return carry

        lax.fori_loop(0, zrows, zb, 0)
        for p in range(nz):
            pltpu.sync_copy(
                zbuf, accum.at[pl.ds(s * rows_per_tile + p * zrows, zrows)])
        plsc.subcore_barrier()

        def body(i, carry):
            off = base + i * chunk
            pltpu.sync_copy(row_hbm.at[pl.ds(off, chunk)], idx)
            pltpu.sync_copy(w_hbm.at[pl.ds(off, chunk)], wv)
            pltpu.sync_copy(wv, accum.at[idx], add=True)
            return carry

        lax.fori_loop(0, n_chunks, body, 0)
        plsc.subcore_barrier()

        pltpu.sync_copy(
            accum.at[pl.ds(s * rows_per_tile, rows_per_tile)],
            out_hbm.at[pl.ds(c * n_nodes + s * rows_per_tile,
                             rows_per_tile)])

    return k


def _node_body(p_ref, x_ref, v_ref, nw1_ref, nb1_ref, nw2_ref, nb2_ref,
               gw_ref, gb_ref, xo_ref, vo_ref):
    m = p_ref[0] + p_ref[1]
    nh = jax.nn.relu(
        jnp.dot(m, nw1_ref[...], preferred_element_type=jnp.float32)
        + nb1_ref[...])
    xo = x_ref[...] + jnp.dot(nh, nw2_ref[...],
                              preferred_element_type=jnp.float32) \
        + nb2_ref[...]
    g = jax.nn.sigmoid(
        jnp.dot(xo, gw_ref[...], preferred_element_type=jnp.float32)
        + gb_ref[...])
    v = v_ref[...]
    xo_ref[...] = xo
    vo_ref[...] = jnp.concatenate(
        [v[:, 0:16] * g, v[:, 16:32] * g, v[:, 32:48] * g], axis=1)


def _tc_node(p, x, v2, nw1, nb1, nw2, nb2, gw, gb):
    n, d = x.shape
    bn = 2000
    grid = (n // bn,)
    full2 = lambda shp: pl.BlockSpec(shp, lambda i: (0, 0))
    return pl.pallas_call(
        _node_body,
        grid=grid,
        in_specs=[
            pl.BlockSpec((2, bn, d), lambda i: (0, i, 0)),
            pl.BlockSpec((bn, d), lambda i: (i, 0)),
            pl.BlockSpec((bn, 48), lambda i: (i, 0)),
            full2((d, d)), full2((1, d)),
            full2((d, d)), full2((1, d)),
            full2((d, 16)), full2((1, 16)),
        ],
        out_specs=[
            pl.BlockSpec((bn, d), lambda i: (i, 0)),
            pl.BlockSpec((bn, 48), lambda i: (i, 0)),
        ],
        out_shape=[
            jax.ShapeDtypeStruct((n, d), jnp.float32),
            jax.ShapeDtypeStruct((n, 48), jnp.float32),
        ],
    )(p, x, v2, nw1, nb1, nw2, nb2, gw, gb)


def kernel(x, M, V, ln_g, ln_b, mW1, mb1, mW2, mb2, nW1, nb1, nW2, nb2,
           iW1, ib1, iW2, ib2, gW, gb, edge_index):
    n, d = x.shape
    e = M.shape[0]
    nw = V.shape[2]
    v2 = V.reshape(n, 3 * nw)
    row = edge_index[0]
    col = edge_index[1]

    t = _node_table(v2)
    g = _make_sc_gather(n, e)(t, row, col)
    m_out, wm = _tc_edge(
        M, g,
        ln_g.reshape(1, d), ln_b.reshape(1, d),
        mW1, mb1.reshape(1, d), mW2, mb2.reshape(1, d),
        iW1, ib1.reshape(1, d), iW2.reshape(1, d), ib2.reshape(1, 1))
    partials = _make_sc_scatter(n, e, d)(wm, row)
    x_out, v2_out = _tc_node(
        partials.reshape(2, n, d), x, v2,
        nW1, nb1.reshape(1, d), nW2, nb2.reshape(1, d),
        gW, gb.reshape(1, 16))
    return (x_out, m_out, v2_out.reshape(n, 3, nw))


# speedup vs baseline: 7.5775x; 1.1648x over previous
import functools

import jax
import jax.numpy as jnp
from jax import lax
from jax.experimental import pallas as pl
from jax.experimental.pallas import tpu as pltpu
from jax.experimental.pallas import tpu_sc as plsc

NC = 2
NS = 16
LANES = 16


def _table_body(v_ref, t_ref):
    v = v_ref[...]
    n = jnp.sqrt(v[:, 0:16] ** 2 + v[:, 16:32] ** 2 + v[:, 32:48] ** 2)
    t_ref[...] = jnp.concatenate([v, n], axis=1)


def _node_table(v2):
    n = v2.shape[0]
    return pl.pallas_call(
        _table_body,
        out_shape=jax.ShapeDtypeStruct((n, 64), jnp.float32),
    )(v2)


def _make_sc_gather(n_nodes, n_edges):
    nwk = NC * NS
    ew = n_edges // nwk
    chunk = 80
    n_chunks = ew // chunk
    mesh = plsc.VectorSubcoreMesh(core_axis_name="c", subcore_axis_name="s", num_cores=NC, num_subcores=NS)

    n_pairs = (n_chunks - 1) // 2
    assert n_chunks == 2 * n_pairs + 1

    @functools.partial(
        pl.kernel,
        out_type=jax.ShapeDtypeStruct((n_edges, 128), jnp.float32),
        mesh=mesh,
        scratch_types=[
            pltpu.VMEM((chunk,), jnp.int32),
            pltpu.VMEM((chunk,), jnp.int32),
            pltpu.VMEM((chunk,), jnp.int32),
            pltpu.VMEM((chunk,), jnp.int32),
            pltpu.VMEM((chunk, 64), jnp.float32),
            pltpu.VMEM((chunk, 64), jnp.float32),
            pltpu.VMEM((chunk, 64), jnp.float32),
            pltpu.VMEM((chunk, 64), jnp.float32),
            pltpu.SemaphoreType.DMA,
            pltpu.SemaphoreType.DMA,
            pltpu.SemaphoreType.DMA,
            pltpu.SemaphoreType.DMA,
            pltpu.SemaphoreType.DMA,
            pltpu.SemaphoreType.DMA,
        ],
        compiler_params=pltpu.CompilerParams(use_tc_tiling_on_sc=False),
    )
    def k(t_hbm, row_hbm, col_hbm, g_hbm,
          ir0, ic0, ir1, ic1, gr0, gc0, gr1, gc1,
          isem0, isem1, gsem0, gsem1, wsem0, wsem1):
        wid = lax.axis_index("s") * NC + lax.axis_index("c")
        base = wid * ew
        ir, ic = (ir0, ir1), (ic0, ic1)
        gr, gc = (gr0, gr1), (gc0, gc1)
        isem, gsem, wsem = (isem0, isem1), (gsem0, gsem1), (wsem0, wsem1)

        def idx_start(j, b):
            off = base + j * chunk
            pltpu.async_copy(row_hbm.at[pl.ds(off, chunk)], ir[b], isem[b])
            pltpu.async_copy(col_hbm.at[pl.ds(off, chunk)], ic[b], isem[b])

        def idx_wait(b):
            pltpu.make_async_copy(
                row_hbm.at[pl.ds(0, chunk)], ir[b], isem[b]).wait()
            pltpu.make_async_copy(
                col_hbm.at[pl.ds(0, chunk)], ic[b], isem[b]).wait()

        def gat_start(b):
            pltpu.async_copy(t_hbm.at[ir[b]], gr[b], gsem[b])
            pltpu.async_copy(t_hbm.at[ic[b]], gc[b], gsem[b])

        def gat_wait(b):
            pltpu.make_async_copy(t_hbm.at[ir[b]], gr[b], gsem[b]).wait()
            pltpu.make_async_copy(t_hbm.at[ic[b]], gc[b], gsem[b]).wait()

        def wr_start(j, b):
            off = base + j * chunk
            pltpu.async_copy(
                gr[b], g_hbm.at[pl.ds(off, chunk), pl.ds(0, 64)], wsem[b])
            pltpu.async_copy(
                gc[b], g_hbm.at[pl.ds(off, chunk), pl.ds(64, 64)], wsem[b])

        def wr_wait(b):
            pltpu.make_async_copy(
                gr[b], g_hbm.at[pl.ds(0, chunk), pl.ds(0, 64)],
                wsem[b]).wait()
            pltpu.make_async_copy(
                gc[b], g_hbm.at[pl.ds(0, chunk), pl.ds(64, 64)],
                wsem[b]).wait()

        idx_start(0, 0)
        idx_wait(0)
        gat_start(0)
        idx_start(1, 1)

        def pair(p, carry):
            for b in range(2):
                j = 2 * p + b
                nb = 1 - b
                gat_wait(b)
                wr_start(j, b)
                idx_wait(nb)

                @pl.when(j >= 1)
                def _():
                    wr_wait(nb)

                gat_start(nb)

                @pl.when(j + 2 < n_chunks)
                def _():
                    idx_start(j + 2, b)

            return carry

        lax.fori_loop(0, n_pairs, pair, 0)

        gat_wait(0)
        wr_wait(1)
        wr_start(n_chunks - 1, 0)
        wr_wait(0)

    return k


def _edge_body(m_ref, g_ref, ln_g_ref, ln_b_ref, w1_ref, b1_ref,
               w2_ref, b2_ref, wab_ref, crep_ref,
               ib1_ref, iw2_ref, ib2_ref, mo_ref, wm_ref):
    m = m_ref[...]
    d = m.shape[1]
    jm = jnp.full((d, d), 1.0 / d, dtype=jnp.float32)
    mu = jnp.dot(m, jm, preferred_element_type=jnp.float32)
    ms = jnp.dot(m * m, jm, preferred_element_type=jnp.float32)
    var = ms - mu * mu
    mn = (m - mu) * lax.rsqrt(var + 1e-5) * ln_g_ref[...] + ln_b_ref[...]
    h = jax.nn.relu(
        jnp.dot(mn, w1_ref[...], preferred_element_type=jnp.float32)
        + b1_ref[...])
    mo = mn + jnp.dot(h, w2_ref[...], preferred_element_type=jnp.float32) \
        + b2_ref[...]

    g = g_ref[...]
    u = g[:, 0:64] * g[:, 64:128]
    dot = u[:, 0:16] + u[:, 16:32] + u[:, 32:48]
    cos = dot / (u[:, 48:64] + 1e-8)
    ih = jax.nn.relu(
        jnp.dot(g, wab_ref[...], preferred_element_type=jnp.float32)
        + jnp.dot(cos, crep_ref[...], preferred_element_type=jnp.float32)
        + ib1_ref[...])
    alpha = jax.nn.sigmoid(
        jnp.sum(ih * iw2_ref[...], axis=1, keepdims=True) + ib2_ref[...])
    mo_ref[...] = mo
    wm_ref[...] = mo * alpha


def _tc_edge(m, g, ln_g, ln_b, w1, b1, w2, b2, iw1, ib1, iw2t, ib2):
    e, d = m.shape
    be = 2000
    grid = (e // be,)
    full = lambda shp: pl.BlockSpec(shp, lambda i: (0, 0))

    wab = jnp.zeros((d, d), jnp.float32)
    wab = wab.at[48:64].set(iw1[0:16]).at[112:128].set(iw1[16:32])
    crep = iw1[32:48]

    return pl.pallas_call(
        _edge_body,
        grid=grid,
        in_specs=[
            pl.BlockSpec((be, d), lambda i: (i, 0)),
            pl.BlockSpec((be, 128), lambda i: (i, 0)),
            full((1, d)), full((1, d)),
            full((d, d)), full((1, d)),
            full((d, d)), full((1, d)),
            full((d, d)), full((16, d)),
            full((1, d)),
            full((1, d)), full((1, 1)),
        ],
        out_specs=[
            pl.BlockSpec((be, d), lambda i: (i, 0)),
            pl.BlockSpec((be, d), lambda i: (i, 0)),
        ],
        out_shape=[
            jax.ShapeDtypeStruct((e, d), jnp.float32),
            jax.ShapeDtypeStruct((e, d), jnp.float32),
        ],
    )(m, g, ln_g, ln_b, w1, b1, w2, b2, wab, crep, ib1, iw2t, ib2)


def _make_sc_scatter(n_nodes, n_edges, d):
    nwk = NC * NS
    ew = n_edges // nwk
    chunk = 80
    n_chunks = ew // chunk
    rows_per_tile = n_nodes // NS
    zrows = 125
    nz = rows_per_tile // zrows
    mesh = plsc.VectorSubcoreMesh(core_axis_name="c", subcore_axis_name="s", num_cores=NC, num_subcores=NS)

    @functools.partial(
        pl.kernel,
        out_type=jax.ShapeDtypeStruct((NC * n_nodes, d), jnp.float32),
        mesh=mesh,
        scratch_types=[
            pltpu.VMEM((chunk,), jnp.int32),
            pltpu.VMEM((chunk, d), jnp.float32),
            pltpu.VMEM((zrows, d), jnp.float32),
            pltpu.VMEM_SHARED((n_nodes, d), jnp.float32),
        ],
        compiler_params=pltpu.CompilerParams(use_tc_tiling_on_sc=False),
    )
    def k(w_hbm, row_hbm, out_hbm, idx, wv, zbuf, accum):
        c = lax.axis_index("c")
        s = lax.axis_index("s")
        wid = s * NC + c
        base = wid * ew

        def zb(i, carry):
            for j in range(d // LANES):
                zbuf[i, pl.ds(j * LANES, LANES)] = jnp.zeros(
                    (LANES,), jnp.float32)
            return carry

        lax.fori_loop(0, zrows, zb, 0)
        for p in range(nz):
            pltpu.sync_copy(
                zbuf, accum.at[pl.ds(s * rows_per_tile + p * zrows, zrows)])
        plsc.subcore_barrier()

        def body(i, carry):
            off = base + i * chunk
            pltpu.sync_copy(row_hbm.at[pl.ds(off, chunk)], idx)
            pltpu.sync_copy(w_hbm.at[pl.ds(off, chunk)], wv)
            pltpu.sync_copy(wv, accum.at[idx], add=True)
            return carry

        lax.fori_loop(0, n_chunks, body, 0)
        plsc.subcore_barrier()

        pltpu.sync_copy(
            accum.at[pl.ds(s * rows_per_tile, rows_per_tile)],
            out_hbm.at[pl.ds(c * n_nodes + s * rows_per_tile,
                             rows_per_tile)])

    return k


def _node_body(p_ref, x_ref, v_ref, nw1_ref, nb1_ref, nw2_ref, nb2_ref,
               gw_ref, gb_ref, xo_ref, vo_ref):
    m = p_ref[0] + p_ref[1]
    nh = jax.nn.relu(
        jnp.dot(m, nw1_ref[...], preferred_element_type=jnp.float32)
        + nb1_ref[...])
    xo = x_ref[...] + jnp.dot(nh, nw2_ref[...],
                              preferred_element_type=jnp.float32) \
        + nb2_ref[...]
    g = jax.nn.sigmoid(
        jnp.dot(xo, gw_ref[...], preferred_element_type=jnp.float32)
        + gb_ref[...])
    v = v_ref[...]
    xo_ref[...] = xo
    vo_ref[...] = jnp.concatenate(
        [v[:, 0:16] * g, v[:, 16:32] * g, v[:, 32:48] * g], axis=1)


def _tc_node(p, x, v2, nw1, nb1, nw2, nb2, gw, gb):
    n, d = x.shape
    bn = 2000
    grid = (n // bn,)
    full2 = lambda shp: pl.BlockSpec(shp, lambda i: (0, 0))
    return pl.pallas_call(
        _node_body,
        grid=grid,
        in_specs=[
            pl.BlockSpec((2, bn, d), lambda i: (0, i, 0)),
            pl.BlockSpec((bn, d), lambda i: (i, 0)),
            pl.BlockSpec((bn, 48), lambda i: (i, 0)),
            full2((d, d)), full2((1, d)),
            full2((d, d)), full2((1, d)),
            full2((d, 16)), full2((1, 16)),
        ],
        out_specs=[
            pl.BlockSpec((bn, d), lambda i: (i, 0)),
            pl.BlockSpec((bn, 48), lambda i: (i, 0)),
        ],
        out_shape=[
            jax.ShapeDtypeStruct((n, d), jnp.float32),
            jax.ShapeDtypeStruct((n, 48), jnp.float32),
        ],
    )(p, x, v2, nw1, nb1, nw2, nb2, gw, gb)


def kernel(x, M, V, ln_g, ln_b, mW1, mb1, mW2, mb2, nW1, nb1, nW2, nb2,
           iW1, ib1, iW2, ib2, gW, gb, edge_index):
    n, d = x.shape
    e = M.shape[0]
    nw = V.shape[2]
    v2 = V.reshape(n, 3 * nw)
    row = edge_index[0]
    col = edge_index[1]

    t = _node_table(v2)
    g = _make_sc_gather(n, e)(t, row, col)
    m_out, wm = _tc_edge(
        M, g,
        ln_g.reshape(1, d), ln_b.reshape(1, d),
        mW1, mb1.reshape(1, d), mW2, mb2.reshape(1, d),
        iW1, ib1.reshape(1, d), iW2.reshape(1, d), ib2.reshape(1, 1))
    partials = _make_sc_scatter(n, e, d)(wm, row)
    x_out, v2_out = _tc_node(
        partials.reshape(2, n, d), x, v2,
        nW1, nb1.reshape(1, d), nW2, nb2.reshape(1, d),
        gW, gb.reshape(1, 16))
    return (x_out, m_out, v2_out.reshape(n, 3, nw))


# speedup vs baseline: 9.1137x; 1.2027x over previous
import functools

import jax
import jax.numpy as jnp
from jax import lax
from jax.experimental import pallas as pl
from jax.experimental.pallas import tpu as pltpu
from jax.experimental.pallas import tpu_sc as plsc

NC = 2
NS = 16
LANES = 16


def _table_body(v_ref, t_ref):
    v = v_ref[...]
    n = jnp.sqrt(v[:, 0:16] ** 2 + v[:, 16:32] ** 2 + v[:, 32:48] ** 2)
    t_ref[...] = jnp.concatenate([v, n], axis=1)


def _node_table(v2):
    n = v2.shape[0]
    return pl.pallas_call(
        _table_body,
        out_shape=jax.ShapeDtypeStruct((n, 64), jnp.float32),
    )(v2)


def _make_sc_gather(n_nodes, n_edges):
    nwk = NC * NS
    ew = n_edges // nwk
    chunk = 80
    n_chunks = ew // chunk
    mesh = plsc.VectorSubcoreMesh(core_axis_name="c", subcore_axis_name="s", num_cores=NC, num_subcores=NS)

    n_pairs = (n_chunks - 1) // 2
    assert n_chunks == 2 * n_pairs + 1

    @functools.partial(
        pl.kernel,
        out_type=jax.ShapeDtypeStruct((n_edges, 128), jnp.float32),
        mesh=mesh,
        scratch_types=[
            pltpu.VMEM((chunk,), jnp.int32),
            pltpu.VMEM((chunk,), jnp.int32),
            pltpu.VMEM((chunk,), jnp.int32),
            pltpu.VMEM((chunk,), jnp.int32),
            pltpu.VMEM((chunk, 64), jnp.float32),
            pltpu.VMEM((chunk, 64), jnp.float32),
            pltpu.VMEM((chunk, 64), jnp.float32),
            pltpu.VMEM((chunk, 64), jnp.float32),
            pltpu.SemaphoreType.DMA,
            pltpu.SemaphoreType.DMA,
            pltpu.SemaphoreType.DMA,
            pltpu.SemaphoreType.DMA,
            pltpu.SemaphoreType.DMA,
            pltpu.SemaphoreType.DMA,
        ],
        compiler_params=pltpu.CompilerParams(use_tc_tiling_on_sc=False),
    )
    def k(t_hbm, row_hbm, col_hbm, g_hbm,
          ir0, ic0, ir1, ic1, gr0, gc0, gr1, gc1,
          isem0, isem1, gsem0, gsem1, wsem0, wsem1):
        wid = lax.axis_index("s") * NC + lax.axis_index("c")
        base = wid * ew
        ir, ic = (ir0, ir1), (ic0, ic1)
        gr, gc = (gr0, gr1), (gc0, gc1)
        isem, gsem, wsem = (isem0, isem1), (gsem0, gsem1), (wsem0, wsem1)

        def idx_start(j, b):
            off = base + j * chunk
            pltpu.async_copy(row_hbm.at[pl.ds(off, chunk)], ir[b], isem[b])
            pltpu.async_copy(col_hbm.at[pl.ds(off, chunk)], ic[b], isem[b])

        def idx_wait(b):
            pltpu.make_async_copy(
                row_hbm.at[pl.ds(0, chunk)], ir[b], isem[b]).wait()
            pltpu.make_async_copy(
                col_hbm.at[pl.ds(0, chunk)], ic[b], isem[b]).wait()

        def gat_start(b):
            pltpu.async_copy(t_hbm.at[ir[b]], gr[b], gsem[b])
            pltpu.async_copy(t_hbm.at[ic[b]], gc[b], gsem[b])

        def gat_wait(b):
            pltpu.make_async_copy(t_hbm.at[ir[b]], gr[b], gsem[b]).wait()
            pltpu.make_async_copy(t_hbm.at[ic[b]], gc[b], gsem[b]).wait()

        def wr_start(j, b):
            off = base + j * chunk
            pltpu.async_copy(
                gr[b], g_hbm.at[pl.ds(off, chunk), pl.ds(0, 64)], wsem[b])
            pltpu.async_copy(
                gc[b], g_hbm.at[pl.ds(off, chunk), pl.ds(64, 64)], wsem[b])

        def wr_wait(b):
            pltpu.make_async_copy(
                gr[b], g_hbm.at[pl.ds(0, chunk), pl.ds(0, 64)],
                wsem[b]).wait()
            pltpu.make_async_copy(
                gc[b], g_hbm.at[pl.ds(0, chunk), pl.ds(64, 64)],
                wsem[b]).wait()

        idx_start(0, 0)
        idx_wait(0)
        gat_start(0)
        idx_start(1, 1)

        def pair(p, carry):
            for b in range(2):
                j = 2 * p + b
                nb = 1 - b
                gat_wait(b)
                wr_start(j, b)
                idx_wait(nb)

                @pl.when(j >= 1)
                def _():
                    wr_wait(nb)

                gat_start(nb)

                @pl.when(j + 2 < n_chunks)
                def _():
                    idx_start(j + 2, b)

            return carry

        lax.fori_loop(0, n_pairs, pair, 0)

        gat_wait(0)
        wr_wait(1)
        wr_start(n_chunks - 1, 0)
        wr_wait(0)

    return k


def _edge_body(m_ref, g_ref, ln_g_ref, ln_b_ref, w1_ref, b1_ref,
               w2_ref, b2_ref, wab_ref, crep_ref,
               ib1_ref, iw2_ref, ib2_ref, mo_ref, wm_ref):
    m = m_ref[...]
    d = m.shape[1]
    jm = jnp.full((d, d), 1.0 / d, dtype=jnp.float32)
    mu = jnp.dot(m, jm, preferred_element_type=jnp.float32)
    ms = jnp.dot(m * m, jm, preferred_element_type=jnp.float32)
    var = ms - mu * mu
    mn = (m - mu) * lax.rsqrt(var + 1e-5) * ln_g_ref[...] + ln_b_ref[...]
    h = jax.nn.relu(
        jnp.dot(mn, w1_ref[...], preferred_element_type=jnp.float32)
        + b1_ref[...])
    mo = mn + jnp.dot(h, w2_ref[...], preferred_element_type=jnp.float32) \
        + b2_ref[...]

    g = g_ref[...]
    u = g[:, 0:64] * g[:, 64:128]
    dot = u[:, 0:16] + u[:, 16:32] + u[:, 32:48]
    cos = dot / (u[:, 48:64] + 1e-8)
    ih = jax.nn.relu(
        jnp.dot(g, wab_ref[...], preferred_element_type=jnp.float32)
        + jnp.dot(cos, crep_ref[...], preferred_element_type=jnp.float32)
        + ib1_ref[...])
    alpha = jax.nn.sigmoid(
        jnp.sum(ih * iw2_ref[...], axis=1, keepdims=True) + ib2_ref[...])
    mo_ref[...] = mo
    wm_ref[...] = mo * alpha


def _tc_edge(m, g, ln_g, ln_b, w1, b1, w2, b2, iw1, ib1, iw2t, ib2):
    e, d = m.shape
    be = 2000
    grid = (e // be,)
    full = lambda shp: pl.BlockSpec(shp, lambda i: (0, 0))

    wab = jnp.zeros((d, d), jnp.float32)
    wab = wab.at[48:64].set(iw1[0:16]).at[112:128].set(iw1[16:32])
    crep = iw1[32:48]

    return pl.pallas_call(
        _edge_body,
        grid=grid,
        in_specs=[
            pl.BlockSpec((be, d), lambda i: (i, 0)),
            pl.BlockSpec((be, 128), lambda i: (i, 0)),
            full((1, d)), full((1, d)),
            full((d, d)), full((1, d)),
            full((d, d)), full((1, d)),
            full((d, d)), full((16, d)),
            full((1, d)),
            full((1, d)), full((1, 1)),
        ],
        out_specs=[
            pl.BlockSpec((be, d), lambda i: (i, 0)),
            pl.BlockSpec((be, d), lambda i: (i, 0)),
        ],
        out_shape=[
            jax.ShapeDtypeStruct((e, d), jnp.float32),
            jax.ShapeDtypeStruct((e, d), jnp.float32),
        ],
    )(m, g, ln_g, ln_b, w1, b1, w2, b2, wab, crep, ib1, iw2t, ib2)


def _make_sc_scatter(n_nodes, n_edges, d):
    nwk = NC * NS
    ew = n_edges // nwk
    chunk = 80
    n_chunks = ew // chunk
    rows_per_tile = n_nodes // NS
    zrows = 125
    nz = rows_per_tile // zrows
    mesh = plsc.VectorSubcoreMesh(core_axis_name="c", subcore_axis_name="s", num_cores=NC, num_subcores=NS)

    n_pairs = (n_chunks - 1) // 2
    assert n_chunks == 2 * n_pairs + 1

    @functools.partial(
        pl.kernel,
        out_type=jax.ShapeDtypeStruct((NC * n_nodes, d), jnp.float32),
        mesh=mesh,
        scratch_types=[
            pltpu.VMEM((chunk,), jnp.int32),
            pltpu.VMEM((chunk,), jnp.int32),
            pltpu.VMEM((chunk, d), jnp.float32),
            pltpu.VMEM((chunk, d), jnp.float32),
            pltpu.VMEM((zrows, d), jnp.float32),
            pltpu.VMEM_SHARED((n_nodes, d), jnp.float32),
            pltpu.SemaphoreType.DMA,
            pltpu.SemaphoreType.DMA,
        ],
        compiler_params=pltpu.CompilerParams(use_tc_tiling_on_sc=False),
    )
    def k(w_hbm, row_hbm, out_hbm, idx0, idx1, wv0, wv1, zbuf, accum,
          lsem0, lsem1):
        c = lax.axis_index("c")
        s = lax.axis_index("s")
        wid = s * NC + c
        base = wid * ew
        idx, wv, lsem = (idx0, idx1), (wv0, wv1), (lsem0, lsem1)

        def lstart(j, b):
            off = base + j * chunk
            pltpu.async_copy(row_hbm.at[pl.ds(off, chunk)], idx[b], lsem[b])
            pltpu.async_copy(w_hbm.at[pl.ds(off, chunk)], wv[b], lsem[b])

        def lwait(b):
            pltpu.make_async_copy(
                row_hbm.at[pl.ds(0, chunk)], idx[b], lsem[b]).wait()
            pltpu.make_async_copy(
                w_hbm.at[pl.ds(0, chunk)], wv[b], lsem[b]).wait()

        lstart(0, 0)
        lstart(1, 1)

        def zb(i, carry):
            for j in range(d // LANES):
                zbuf[i, pl.ds(j * LANES, LANES)] = jnp.zeros(
                    (LANES,), jnp.float32)
            return carry

        lax.fori_loop(0, zrows, zb, 0)
        for p in range(nz):
            pltpu.sync_copy(
                zbuf, accum.at[pl.ds(s * rows_per_tile + p * zrows, zrows)])
        plsc.subcore_barrier()

        def pair(p, carry):
            for b in range(2):
                j = 2 * p + b
                lwait(b)
                pltpu.sync_copy(wv[b], accum.at[idx[b]], add=True)

                @pl.when(j + 2 < n_chunks)
                def _():
                    lstart(j + 2, b)

            return carry

        lax.fori_loop(0, n_pairs, pair, 0)
        lwait(0)
        pltpu.sync_copy(wv[0], accum.at[idx[0]], add=True)
        plsc.subcore_barrier()

        pltpu.sync_copy(
            accum.at[pl.ds(s * rows_per_tile, rows_per_tile)],
            out_hbm.at[pl.ds(c * n_nodes + s * rows_per_tile,
                             rows_per_tile)])

    return k


def _node_body(p_ref, x_ref, v_ref, nw1_ref, nb1_ref, nw2_ref, nb2_ref,
               gw_ref, gb_ref, xo_ref, vo_ref):
    m = p_ref[0] + p_ref[1]
    nh = jax.nn.relu(
        jnp.dot(m, nw1_ref[...], preferred_element_type=jnp.float32)
        + nb1_ref[...])
    xo = x_ref[...] + jnp.dot(nh, nw2_ref[...],
                              preferred_element_type=jnp.float32) \
        + nb2_ref[...]
    g = jax.nn.sigmoid(
        jnp.dot(xo, gw_ref[...], preferred_element_type=jnp.float32)
        + gb_ref[...])
    v = v_ref[...]
    xo_ref[...] = xo
    vo_ref[...] = jnp.concatenate(
        [v[:, 0:16] * g, v[:, 16:32] * g, v[:, 32:48] * g], axis=1)


def _tc_node(p, x, v2, nw1, nb1, nw2, nb2, gw, gb):
    n, d = x.shape
    bn = 2000
    grid = (n // bn,)
    full2 = lambda shp: pl.BlockSpec(shp, lambda i: (0, 0))
    return pl.pallas_call(
        _node_body,
        grid=grid,
        in_specs=[
            pl.BlockSpec((2, bn, d), lambda i: (0, i, 0)),
            pl.BlockSpec((bn, d), lambda i: (i, 0)),
            pl.BlockSpec((bn, 48), lambda i: (i, 0)),
            full2((d, d)), full2((1, d)),
            full2((d, d)), full2((1, d)),
            full2((d, 16)), full2((1, 16)),
        ],
        out_specs=[
            pl.BlockSpec((bn, d), lambda i: (i, 0)),
            pl.BlockSpec((bn, 48), lambda i: (i, 0)),
        ],
        out_shape=[
            jax.ShapeDtypeStruct((n, d), jnp.float32),
            jax.ShapeDtypeStruct((n, 48), jnp.float32),
        ],
    )(p, x, v2, nw1, nb1, nw2, nb2, gw, gb)


def kernel(x, M, V, ln_g, ln_b, mW1, mb1, mW2, mb2, nW1, nb1, nW2, nb2,
           iW1, ib1, iW2, ib2, gW, gb, edge_index):
    n, d = x.shape
    e = M.shape[0]
    nw = V.shape[2]
    v2 = V.reshape(n, 3 * nw)
    row = edge_index[0]
    col = edge_index[1]

    t = _node_table(v2)
    g = _make_sc_gather(n, e)(t, row, col)
    m_out, wm = _tc_edge(
        M, g,
        ln_g.reshape(1, d), ln_b.reshape(1, d),
        mW1, mb1.reshape(1, d), mW2, mb2.reshape(1, d),
        iW1, ib1.reshape(1, d), iW2.reshape(1, d), ib2.reshape(1, 1))
    partials = _make_sc_scatter(n, e, d)(wm, row)
    x_out, v2_out = _tc_node(
        partials.reshape(2, n, d), x, v2,
        nW1, nb1.reshape(1, d), nW2, nb2.reshape(1, d),
        gW, gb.reshape(1, 16))
    return (x_out, m_out, v2_out.reshape(n, 3, nw))
